# baseline (speedup 1.0000x reference)
import functools

import jax
import jax.numpy as jnp
from jax import lax
from jax.experimental import pallas as pl
from jax.experimental.pallas import tpu as pltpu
from jax.experimental.pallas import tpu_sc as plsc

_EMB = 64
_B = 16384
_TL = 20
_DL = 200

_NC = 2
_NS = 16
_NW = _NC * _NS

_IW = 128
_SPW = 4
_CW = _IW * _SPW
_NBUF = 2

_T_TOTAL = _B * _TL
_D_TOTAL = _B * _DL


def _gather_table(tbl, idxT, out_hbm, idx_v, rows_v, sem_i, sem_g, sem_w,
                  wid, n_words):
    col0 = wid * _CW

    def idx_src(j):
        return idxT.at[j, pl.ds(col0, _CW)]

    def out_dst(j):
        return out_hbm.at[pl.ds(j * _B + col0, _CW)]

    def fire_gathers(b):
        for s in range(_SPW):
            pltpu.async_copy(tbl.at[idx_v.at[b, pl.ds(s * _IW, _IW)]],
                             rows_v.at[b, pl.ds(s * _IW, _IW)], sem_g[b])

    def wait_gathers(b):
        for s in range(_SPW):
            pltpu.make_async_copy(tbl.at[idx_v.at[b, pl.ds(s * _IW, _IW)]],
                                  rows_v.at[b, pl.ds(s * _IW, _IW)],
                                  sem_g[b]).wait()

    def wait_idx(b):
        pltpu.make_async_copy(idx_src(0), idx_v.at[b], sem_i[b]).wait()

    def wait_wb(b):
        pltpu.make_async_copy(rows_v.at[b], out_dst(0), sem_w[b]).wait()

    n_groups = n_words // _NBUF
    for b in range(_NBUF):
        pltpu.async_copy(idx_src(b), idx_v.at[b], sem_i[b])

    @pl.loop(0, n_groups)
    def _group(gi):
        j0 = gi * _NBUF
        for b in range(_NBUF):
            wait_idx(b)

            @pl.when(gi > 0)
            def _():
                wait_wb(b)

            fire_gathers(b)
        for b in range(_NBUF):
            wait_gathers(b)
            pltpu.async_copy(rows_v.at[b], out_dst(j0 + b), sem_w[b])

            @pl.when(gi < n_groups - 1)
            def _():
                pltpu.async_copy(idx_src(j0 + _NBUF + b), idx_v.at[b],
                                 sem_i[b])

    for b in range(_NBUF):
        wait_wb(b)


def _sc_body(t_idxT, d_idxT, t_tbl, d_tbl, out_t, out_d, idx_v, rows_v,
             si0, si1, sg0, sg1, sw0, sw1):
    wid = lax.axis_index("s") * _NC + lax.axis_index("c")
    sem_i = (si0, si1)
    sem_g = (sg0, sg1)
    sem_w = (sw0, sw1)
    _gather_table(t_tbl, t_idxT, out_t, idx_v, rows_v, sem_i, sem_g, sem_w,
                  wid, _TL)
    _gather_table(d_tbl, d_idxT, out_d, idx_v, rows_v, sem_i, sem_g, sem_w,
                  wid, _DL)


def _sc_gather(t_idxT, d_idxT, t_tbl, d_tbl):
    mesh = plsc.VectorSubcoreMesh(core_axis_name="c", subcore_axis_name="s")
    return pl.kernel(
        _sc_body,
        out_type=(
            jax.ShapeDtypeStruct((_T_TOTAL, _EMB), jnp.float32),
            jax.ShapeDtypeStruct((_D_TOTAL, _EMB), jnp.float32),
        ),
        mesh=mesh,
        scratch_types=[
            pltpu.VMEM((_NBUF, _CW), jnp.int32),
            pltpu.VMEM((_NBUF, _CW, _EMB), jnp.float32),
        ] + [pltpu.SemaphoreType.DMA] * 6,
        compiler_params=pltpu.CompilerParams(use_tc_tiling_on_sc=False),
    )(t_idxT, d_idxT, t_tbl, d_tbl)


def _tc_body(x_ref, o_ref):
    for q in range(16):
        x = x_ref[0, pl.ds(q * 64, 64), :]
        y = jnp.transpose(x.reshape(64, 2, _EMB), (2, 0, 1))
        o_ref[0, :, pl.ds(q * 8, 8), :] = y.reshape(8, 8, 128)


def _tc_detile(flat, n_words):
    x3 = flat.reshape(n_words, _B // 2, 128)
    return pl.pallas_call(
        _tc_body,
        grid=(n_words, 8),
        in_specs=[pl.BlockSpec((1, 1024, 128), lambda j, g: (j, g, 0))],
        out_specs=pl.BlockSpec((1, 8, 128, 128),
                               lambda j, g: (j, 0, g, 0)),
        out_shape=jax.ShapeDtypeStruct((n_words, 8, 1024, 128), jnp.float32),
    )(x3)


def _relabel(o4, n_words):
    o5 = o4.reshape(n_words, 8, 128, 8, 128)
    return o5.transpose(2, 4, 0, 1, 3).reshape(_B, n_words, _EMB)


@jax.jit
def _lookup(title, description, title_table, description_table):
    t_idxT = title.astype(jnp.int32).T
    d_idxT = description.astype(jnp.int32).T
    out_t, out_d = _sc_gather(t_idxT, d_idxT, title_table, description_table)
    o4_t = _tc_detile(out_t.reshape(_T_TOTAL // 2, 2 * _EMB), _TL)
    o4_d = _tc_detile(out_d.reshape(_D_TOTAL // 2, 2 * _EMB), _DL)
    return (_relabel(o4_t, _TL), _relabel(o4_d, _DL))


def kernel(title, description, title_table, description_table):
    return _lookup(title, description, title_table, description_table)


# speedup vs baseline: 13.5639x; 13.5639x over previous
import functools

import jax
import jax.numpy as jnp
from jax import lax
from jax.experimental import pallas as pl
from jax.experimental.pallas import tpu as pltpu
from jax.experimental.pallas import tpu_sc as plsc

_EMB = 64
_B = 16384
_TL = 20
_DL = 200

_NC = 2
_NS = 16
_NW = _NC * _NS

_IW = 128
_SPW = 4
_CW = _IW * _SPW
_NBUF = 2

_T_TOTAL = _B * _TL
_D_TOTAL = _B * _DL


def _gather_table(tbl, idxT, out_hbm, idx_v, rows_v, sem_i, sem_g, sem_w,
                  wid, n_words):
    col0 = wid * _CW

    def idx_src(j):
        return idxT.at[j, pl.ds(col0, _CW)]

    def out_dst(j2, b):
        return out_hbm.at[j2, pl.ds(col0, _CW), pl.ds(b * _EMB, _EMB)]

    def fire_gathers(b):
        for s in range(_SPW):
            pltpu.async_copy(tbl.at[idx_v.at[b, pl.ds(s * _IW, _IW)]],
                             rows_v.at[b, pl.ds(s * _IW, _IW)], sem_g[b])

    def wait_gathers(b):
        for s in range(_SPW):
            pltpu.make_async_copy(tbl.at[idx_v.at[b, pl.ds(s * _IW, _IW)]],
                                  rows_v.at[b, pl.ds(s * _IW, _IW)],
                                  sem_g[b]).wait()

    def wait_idx(b):
        pltpu.make_async_copy(idx_src(0), idx_v.at[b], sem_i[b]).wait()

    def wait_wb(b):
        pltpu.make_async_copy(rows_v.at[b], out_dst(0, b), sem_w[b]).wait()

    n_groups = n_words // _NBUF
    for b in range(_NBUF):
        pltpu.async_copy(idx_src(b), idx_v.at[b], sem_i[b])

    @pl.loop(0, n_groups)
    def _group(gi):
        j0 = gi * _NBUF
        for b in range(_NBUF):
            wait_idx(b)

            @pl.when(gi > 0)
            def _():
                wait_wb(b)

            fire_gathers(b)
        for b in range(_NBUF):
            wait_gathers(b)
            pltpu.async_copy(rows_v.at[b], out_dst(gi, b), sem_w[b])

            @pl.when(gi < n_groups - 1)
            def _():
                pltpu.async_copy(idx_src(j0 + _NBUF + b), idx_v.at[b],
                                 sem_i[b])

    for b in range(_NBUF):
        wait_wb(b)


def _sc_body(t_idxT, d_idxT, t_tbl, d_tbl, out_t, out_d, idx_v, rows_v,
             si0, si1, sg0, sg1, sw0, sw1):
    wid = lax.axis_index("s") * _NC + lax.axis_index("c")
    sem_i = (si0, si1)
    sem_g = (sg0, sg1)
    sem_w = (sw0, sw1)
    _gather_table(t_tbl, t_idxT, out_t, idx_v, rows_v, sem_i, sem_g, sem_w,
                  wid, _TL)
    _gather_table(d_tbl, d_idxT, out_d, idx_v, rows_v, sem_i, sem_g, sem_w,
                  wid, _DL)


def _sc_gather(t_idxT, d_idxT, t_tbl, d_tbl):
    mesh = plsc.VectorSubcoreMesh(core_axis_name="c", subcore_axis_name="s")
    return pl.kernel(
        _sc_body,
        out_type=(
            jax.ShapeDtypeStruct((_TL // 2, _B, 2 * _EMB), jnp.float32),
            jax.ShapeDtypeStruct((_DL // 2, _B, 2 * _EMB), jnp.float32),
        ),
        mesh=mesh,
        scratch_types=[
            pltpu.VMEM((_NBUF, _CW), jnp.int32),
            pltpu.VMEM((_NBUF, _CW, _EMB), jnp.float32),
        ] + [pltpu.SemaphoreType.DMA] * 6,
        compiler_params=pltpu.CompilerParams(use_tc_tiling_on_sc=False),
    )(t_idxT, d_idxT, t_tbl, d_tbl)


def _tc_body(x_ref, o_ref):
    for q in range(16):
        x = x_ref[0, pl.ds(q * 128, 128), :]
        o_ref[:, :, pl.ds(q * 8, 8), :] = x.T.reshape(2, 8, 8, 128)


def _tc_detile(x3, n_words):
    return pl.pallas_call(
        _tc_body,
        grid=(n_words // 2, 8),
        in_specs=[pl.BlockSpec((1, 2048, 128), lambda j2, g: (j2, g, 0))],
        out_specs=pl.BlockSpec((2, 8, 128, 128),
                               lambda j2, g: (j2, 0, g, 0)),
        out_shape=jax.ShapeDtypeStruct((n_words, 8, 1024, 128), jnp.float32),
    )(x3)


def _relabel(o4, n_words):
    o5 = o4.reshape(n_words, 8, 128, 8, 128)
    return o5.transpose(2, 4, 0, 1, 3).reshape(_B, n_words, _EMB)


@jax.jit
def _lookup(title, description, title_table, description_table):
    t_idxT = title.astype(jnp.int32).T
    d_idxT = description.astype(jnp.int32).T
    out_t, out_d = _sc_gather(t_idxT, d_idxT, title_table, description_table)
    o4_t = _tc_detile(out_t, _TL)
    o4_d = _tc_detile(out_d, _DL)
    return (_relabel(o4_t, _TL), _relabel(o4_d, _DL))


def kernel(title, description, title_table, description_table):
    return _lookup(title, description, title_table, description_table)
